# Initial kernel scaffold; baseline (speedup 1.0000x reference)
import jax
import jax.numpy as jnp
from jax.experimental import pallas as pl


def kernel(x, edge_index, edge_time, node_time, edge_weight, W_g, W1, b1, gamma, beta, W2, b2):
    raise NotImplementedError("write your pallas kernel here")


# speedup vs baseline: 7.8077x; 7.8077x over previous
import functools

import jax
import jax.numpy as jnp
from jax import lax
from jax.experimental import pallas as pl
from jax.experimental.pallas import tpu as pltpu
from jax.experimental.pallas import tpu_sc as plsc

N, E, D = 10000, 320000, 128
H1, H2 = 128, 64
NC, NS, L = 2, 16, 16
NW = NC * NS
EPW = E // NW
KB = 80
NB = EPW // KB
RPT = 624
RREM = N - NS * RPT


@functools.partial(
    pl.kernel,
    out_type=jax.ShapeDtypeStruct((NC, N, D), jnp.float32),
    mesh=plsc.VectorSubcoreMesh(
        core_axis_name="c", subcore_axis_name="s", num_cores=NC, num_subcores=NS
    ),
    scratch_types=[
        pltpu.VMEM_SHARED((N, D), jnp.float32),
        pltpu.VMEM((KB,), jnp.int32),
        pltpu.VMEM((KB,), jnp.int32),
        pltpu.VMEM((KB,), jnp.float32),
        pltpu.VMEM((KB,), jnp.float32),
        pltpu.VMEM((KB,), jnp.float32),
        pltpu.VMEM((KB,), jnp.float32),
        pltpu.VMEM((KB, D), jnp.float32),
        pltpu.SemaphoreType.DMA,
    ],
)
def _sc_aggregate(x_hbm, src_hbm, dst_hbm, et_hbm, ew_hbm, nt_hbm, out_hbm,
                  agg_sh, src_v, dst_v, et_v, ew_v, ntd_v, coef_v, rows_v,
                  sem):
    c = lax.axis_index("c")
    s = lax.axis_index("s")
    wid = c * NS + s

    row0 = s * RPT
    pltpu.sync_copy(x_hbm.at[pl.ds(row0, RPT)], agg_sh.at[pl.ds(row0, RPT)])

    @pl.when(s == NS - 1)
    def _init_tail():
        pltpu.sync_copy(x_hbm.at[pl.ds(NS * RPT, RREM)],
                        agg_sh.at[pl.ds(NS * RPT, RREM)])

    plsc.subcore_barrier()

    ebase = wid * EPW

    def batch(b, carry):
        base = ebase + b * KB
        pltpu.sync_copy(src_hbm.at[pl.ds(base, KB)], src_v)
        pltpu.sync_copy(dst_hbm.at[pl.ds(base, KB)], dst_v)
        pltpu.sync_copy(et_hbm.at[pl.ds(base, KB)], et_v)
        pltpu.sync_copy(ew_hbm.at[pl.ds(base, KB)], ew_v)
        cp_rows = pltpu.async_copy(x_hbm.at[src_v], rows_v, sem)
        cp_ntd = pltpu.async_copy(nt_hbm.at[dst_v], ntd_v, sem)
        cp_rows.wait()
        cp_ntd.wait()

        def cgrp(g, carry2):
            nt = ntd_v[pl.ds(g * L, L)]
            etg = et_v[pl.ds(g * L, L)]
            ewg = ew_v[pl.ds(g * L, L)]
            coef_v[pl.ds(g * L, L)] = ewg * jnp.exp(-jnp.abs(nt - etg))
            return carry2

        lax.fori_loop(0, KB // L, cgrp, 0)

        def scale(g, carry2):
            cvec = coef_v[pl.ds(g * L, L)]
            for li in range(L):
                cf = cvec[li]
                e = g * L + li
                for j in range(D // L):
                    rows_v[e, pl.ds(j * L, L)] = rows_v[e, pl.ds(j * L, L)] * cf
            return carry2

        lax.fori_loop(0, KB // L, scale, 0)
        pltpu.sync_copy(rows_v, agg_sh.at[dst_v], add=True)
        return carry

    lax.fori_loop(0, NB, batch, 0)

    plsc.subcore_barrier()
    pltpu.sync_copy(agg_sh.at[pl.ds(row0, RPT)],
                    out_hbm.at[c, pl.ds(row0, RPT)])

    @pl.when(s == NS - 1)
    def _emit_tail():
        pltpu.sync_copy(agg_sh.at[pl.ds(NS * RPT, RREM)],
                        out_hbm.at[c, pl.ds(NS * RPT, RREM)])


def _dense_body(p_ref, x_ref, wg_ref, w1_ref, b1_ref, g_ref, be_ref, w2_ref,
                b2_ref, o_ref):
    agg = p_ref[0] + p_ref[1] - x_ref[...]
    ne = jnp.maximum(jnp.dot(agg, wg_ref[...],
                             preferred_element_type=jnp.float32), 0.0)
    h = jnp.dot(ne, w1_ref[...], preferred_element_type=jnp.float32)
    h = jnp.maximum(h + b1_ref[...], 0.0)
    mean = jnp.mean(h, axis=0, keepdims=True)
    var = jnp.mean((h - mean) * (h - mean), axis=0, keepdims=True)
    hn = (h - mean) / jnp.sqrt(var + 1e-5) * g_ref[...] + be_ref[...]
    out = jnp.dot(hn, w2_ref[...], preferred_element_type=jnp.float32)
    o_ref[...] = jax.nn.sigmoid(out + b2_ref[...])


_dense_call = pl.pallas_call(
    _dense_body,
    out_shape=jax.ShapeDtypeStruct((N, H2), jnp.float32),
)


def kernel(x, edge_index, edge_time, node_time, edge_weight,
           W_g, W1, b1, gamma, beta, W2, b2):
    src = edge_index[0]
    dst = edge_index[1]
    partials = _sc_aggregate(x, src, dst, edge_time, edge_weight, node_time)
    return _dense_call(partials, x, W_g, W1, b1.reshape(1, H1),
                       gamma.reshape(1, H1), beta.reshape(1, H1), W2,
                       b2.reshape(1, H2))


# speedup vs baseline: 13.0314x; 1.6690x over previous
import jax
import jax.numpy as jnp
from jax import lax
from jax.experimental import pallas as pl
from jax.experimental.pallas import tpu as pltpu
from jax.experimental.pallas import tpu_sc as plsc

N, E, D = 10000, 320000, 128
H1, H2 = 128, 64
NC, NS, L = 2, 16, 16
NW = NC * NS
EPW = E // NW
KB = 96
NB = 105
EPW_P = NB * KB
NBUF = 3
RPT = 624
RREM = N - NS * RPT


def _sc_body(x_hbm, pki_hbm, pkf_hbm, nt_hbm, out_hbm, agg_sh,
             pb0, pb1, pb2, pf0, pf1, pf2,
             rw0, rw1, rw2, nd0, nd1, nd2,
             es0, es1, es2, gs0, gs1, gs2, ss0, ss1, ss2):
    pbufs = (pb0, pb1, pb2)
    fbufs = (pf0, pf1, pf2)
    rows = (rw0, rw1, rw2)
    ntds = (nd0, nd1, nd2)
    esems = (es0, es1, es2)
    gsems = (gs0, gs1, gs2)
    ssems = (ss0, ss1, ss2)

    c = lax.axis_index("c")
    s = lax.axis_index("s")
    wid = c * NS + s
    blk0 = wid * NB

    row0 = s * RPT
    pltpu.sync_copy(x_hbm.at[pl.ds(row0, RPT)], agg_sh.at[pl.ds(row0, RPT)])

    @pl.when(s == NS - 1)
    def _init_tail():
        pltpu.sync_copy(x_hbm.at[pl.ds(NS * RPT, RREM)],
                        agg_sh.at[pl.ds(NS * RPT, RREM)])

    plsc.subcore_barrier()

    def edge_copy(b, q):
        pltpu.async_copy(pki_hbm.at[blk0 + b], pbufs[q], esems[q])
        pltpu.async_copy(pkf_hbm.at[blk0 + b], fbufs[q], esems[q])

    def wait_edge(b, q):
        pltpu.make_async_copy(pki_hbm.at[blk0 + b], pbufs[q],
                              esems[q]).wait()
        pltpu.make_async_copy(pkf_hbm.at[blk0 + b], fbufs[q],
                              esems[q]).wait()

    def fire_gathers(q):
        pltpu.async_copy(x_hbm.at[pbufs[q].at[1]], rows[q], gsems[q])
        pltpu.async_copy(nt_hbm.at[pbufs[q].at[0]], ntds[q], gsems[q])

    def wait_gathers(q):
        pltpu.make_async_copy(x_hbm.at[pbufs[q].at[1]], rows[q],
                              gsems[q]).wait()
        pltpu.make_async_copy(nt_hbm.at[pbufs[q].at[0]], ntds[q],
                              gsems[q]).wait()

    def fire_scatter(q):
        pltpu.async_copy(rows[q], agg_sh.at[pbufs[q].at[0]], ssems[q],
                         add=True)

    def wait_scatter(q):
        pltpu.make_async_copy(rows[q], agg_sh.at[pbufs[q].at[0]],
                              ssems[q]).wait()

    def compute(q):
        rq = rows[q]

        def grp(g, carry):
            nt = ntds[q][pl.ds(g * L, L)]
            etv = fbufs[q][0, pl.ds(g * L, L)]
            ewv = fbufs[q][1, pl.ds(g * L, L)]
            cvec = ewv * jnp.exp(-jnp.abs(nt - etv))
            for li in range(L):
                cf = cvec[li]
                e = g * L + li
                for j in range(D // L):
                    rq[e, pl.ds(j * L, L)] = rq[e, pl.ds(j * L, L)] * cf
            return carry

        lax.fori_loop(0, KB // L, grp, 0)

    edge_copy(0, 0)
    edge_copy(1, 1)
    wait_edge(0, 0)
    fire_gathers(0)

    def step(b, q):
        pa = (q + 1) % NBUF
        pp = (q + 2) % NBUF

        @pl.when(b + 1 < NB)
        def _advance():
            wait_edge(b + 1, pa)
            fire_gathers(pa)

        wait_gathers(q)
        compute(q)
        fire_scatter(q)

        @pl.when(b >= 1)
        def _retire():
            wait_scatter(pp)

        @pl.when(b + 2 < NB)
        def _prefetch():
            edge_copy(b + 2, pp)

    def superstep(i, carry):
        for q in range(NBUF):
            step(i * NBUF + q, q)
        return carry

    lax.fori_loop(0, NB // NBUF, superstep, 0)
    wait_scatter((NB - 1) % NBUF)

    plsc.subcore_barrier()
    pltpu.sync_copy(agg_sh.at[pl.ds(row0, RPT)],
                    out_hbm.at[c, pl.ds(row0, RPT)])

    @pl.when(s == NS - 1)
    def _emit_tail():
        pltpu.sync_copy(agg_sh.at[pl.ds(NS * RPT, RREM)],
                        out_hbm.at[c, pl.ds(NS * RPT, RREM)])


_sc_aggregate = pl.kernel(
    _sc_body,
    out_type=jax.ShapeDtypeStruct((NC, N, D), jnp.float32),
    mesh=plsc.VectorSubcoreMesh(
        core_axis_name="c", subcore_axis_name="s", num_cores=NC,
        num_subcores=NS,
    ),
    scratch_types=(
        [pltpu.VMEM_SHARED((N, D), jnp.float32)]
        + [pltpu.VMEM((2, KB), jnp.int32) for _ in range(NBUF)]
        + [pltpu.VMEM((2, KB), jnp.float32) for _ in range(NBUF)]
        + [pltpu.VMEM((KB, D), jnp.float32) for _ in range(NBUF)]
        + [pltpu.VMEM((KB,), jnp.float32) for _ in range(NBUF)]
        + [pltpu.SemaphoreType.DMA for _ in range(3 * NBUF)]
    ),
)


def _dense_body(p_ref, x_ref, wg_ref, w1_ref, b1_ref, g_ref, be_ref, w2_ref,
                b2_ref, o_ref):
    agg = p_ref[0] + p_ref[1] - x_ref[...]
    ne = jnp.maximum(jnp.dot(agg, wg_ref[...],
                             preferred_element_type=jnp.float32), 0.0)
    h = jnp.dot(ne, w1_ref[...], preferred_element_type=jnp.float32)
    h = jnp.maximum(h + b1_ref[...], 0.0)
    mean = jnp.mean(h, axis=0, keepdims=True)
    var = jnp.mean((h - mean) * (h - mean), axis=0, keepdims=True)
    hn = (h - mean) / jnp.sqrt(var + 1e-5) * g_ref[...] + be_ref[...]
    out = jnp.dot(hn, w2_ref[...], preferred_element_type=jnp.float32)
    o_ref[...] = jax.nn.sigmoid(out + b2_ref[...])


_dense_call = pl.pallas_call(
    _dense_body,
    out_shape=jax.ShapeDtypeStruct((N, H2), jnp.float32),
)


def _pack_edges(edge_index, edge_time, edge_weight):
    pad = EPW_P - EPW
    dst = edge_index[1].reshape(NW, EPW)
    src = edge_index[0].reshape(NW, EPW)
    et = edge_time.reshape(NW, EPW)
    ew = edge_weight.reshape(NW, EPW)
    cfg = [(0, 0), (0, pad)]
    dst, src, et, ew = [jnp.pad(a, cfg).reshape(NW, NB, KB)
                        for a in (dst, src, et, ew)]
    pki = jnp.stack([dst, src], axis=2).reshape(NW * NB, 2, KB)
    pkf = jnp.stack([et, ew], axis=2).reshape(NW * NB, 2, KB)
    return pki, pkf


def kernel(x, edge_index, edge_time, node_time, edge_weight,
           W_g, W1, b1, gamma, beta, W2, b2):
    pki, pkf = _pack_edges(edge_index, edge_time, edge_weight)
    partials = _sc_aggregate(x, pki, pkf, node_time)
    return _dense_call(partials, x, W_g, W1, b1.reshape(1, H1),
                       gamma.reshape(1, H1), beta.reshape(1, H1), W2,
                       b2.reshape(1, H2))


# speedup vs baseline: 13.3121x; 1.0215x over previous
import jax
import jax.numpy as jnp
from jax import lax
from jax.experimental import pallas as pl
from jax.experimental.pallas import tpu as pltpu
from jax.experimental.pallas import tpu_sc as plsc

N, E, D = 10000, 320000, 128
H1, H2 = 128, 64
NC, NS, L = 2, 16, 16
NW = NC * NS
EPW = E // NW
KB = 112
NB = 90
EPW_P = NB * KB
NBUF = 3
RPT = 624
RREM = N - NS * RPT


def _sc_body(x_hbm, pki_hbm, pkf_hbm, nt_hbm, out_hbm, agg_sh,
             pb0, pb1, pb2, pf0, pf1, pf2,
             rw0, rw1, rw2, nd0, nd1, nd2,
             es0, es1, es2, gs0, gs1, gs2, ss0, ss1, ss2):
    pbufs = (pb0, pb1, pb2)
    fbufs = (pf0, pf1, pf2)
    rows = (rw0, rw1, rw2)
    ntds = (nd0, nd1, nd2)
    esems = (es0, es1, es2)
    gsems = (gs0, gs1, gs2)
    ssems = (ss0, ss1, ss2)

    c = lax.axis_index("c")
    s = lax.axis_index("s")
    wid = c * NS + s
    blk0 = wid * NB

    row0 = s * RPT
    pltpu.sync_copy(x_hbm.at[pl.ds(row0, RPT)], agg_sh.at[pl.ds(row0, RPT)])

    @pl.when(s == NS - 1)
    def _init_tail():
        pltpu.sync_copy(x_hbm.at[pl.ds(NS * RPT, RREM)],
                        agg_sh.at[pl.ds(NS * RPT, RREM)])

    plsc.subcore_barrier()

    def edge_copy(b, q):
        pltpu.async_copy(pki_hbm.at[blk0 + b], pbufs[q], esems[q])
        pltpu.async_copy(pkf_hbm.at[blk0 + b], fbufs[q], esems[q])

    def wait_edge(b, q):
        pltpu.make_async_copy(pki_hbm.at[blk0 + b], pbufs[q],
                              esems[q]).wait()
        pltpu.make_async_copy(pkf_hbm.at[blk0 + b], fbufs[q],
                              esems[q]).wait()

    def fire_gathers(q):
        pltpu.async_copy(x_hbm.at[pbufs[q].at[1]], rows[q], gsems[q])
        pltpu.async_copy(nt_hbm.at[pbufs[q].at[0]], ntds[q], gsems[q])

    def wait_gathers(q):
        pltpu.make_async_copy(x_hbm.at[pbufs[q].at[1]], rows[q],
                              gsems[q]).wait()
        pltpu.make_async_copy(nt_hbm.at[pbufs[q].at[0]], ntds[q],
                              gsems[q]).wait()

    def fire_scatter(q):
        pltpu.async_copy(rows[q], agg_sh.at[pbufs[q].at[0]], ssems[q],
                         add=True)

    def wait_scatter(q):
        pltpu.make_async_copy(rows[q], agg_sh.at[pbufs[q].at[0]],
                              ssems[q]).wait()

    def compute(q):
        rq = rows[q]

        def grp(g, carry):
            nt = ntds[q][pl.ds(g * L, L)]
            etv = fbufs[q][0, pl.ds(g * L, L)]
            ewv = fbufs[q][1, pl.ds(g * L, L)]
            cvec = ewv * jnp.exp(-jnp.abs(nt - etv))
            for li in range(L):
                cf = cvec[li]
                e = g * L + li
                for j in range(D // L):
                    rq[e, pl.ds(j * L, L)] = rq[e, pl.ds(j * L, L)] * cf
            return carry

        lax.fori_loop(0, KB // L, grp, 0)

    edge_copy(0, 0)
    edge_copy(1, 1)
    wait_edge(0, 0)
    fire_gathers(0)

    def step(b, q):
        pa = (q + 1) % NBUF
        pp = (q + 2) % NBUF

        @pl.when(b + 1 < NB)
        def _advance():
            wait_edge(b + 1, pa)
            fire_gathers(pa)

        wait_gathers(q)
        compute(q)
        fire_scatter(q)

        @pl.when(b >= 1)
        def _retire():
            wait_scatter(pp)

        @pl.when(b + 2 < NB)
        def _prefetch():
            edge_copy(b + 2, pp)

    def superstep(i, carry):
        for q in range(NBUF):
            step(i * NBUF + q, q)
        return carry

    lax.fori_loop(0, NB // NBUF, superstep, 0)
    wait_scatter((NB - 1) % NBUF)

    plsc.subcore_barrier()
    pltpu.sync_copy(agg_sh.at[pl.ds(row0, RPT)],
                    out_hbm.at[c, pl.ds(row0, RPT)])

    @pl.when(s == NS - 1)
    def _emit_tail():
        pltpu.sync_copy(agg_sh.at[pl.ds(NS * RPT, RREM)],
                        out_hbm.at[c, pl.ds(NS * RPT, RREM)])


_sc_aggregate = pl.kernel(
    _sc_body,
    out_type=jax.ShapeDtypeStruct((NC, N, D), jnp.float32),
    mesh=plsc.VectorSubcoreMesh(
        core_axis_name="c", subcore_axis_name="s", num_cores=NC,
        num_subcores=NS,
    ),
    scratch_types=(
        [pltpu.VMEM_SHARED((N, D), jnp.float32)]
        + [pltpu.VMEM((2, KB), jnp.int32) for _ in range(NBUF)]
        + [pltpu.VMEM((2, KB), jnp.float32) for _ in range(NBUF)]
        + [pltpu.VMEM((KB, D), jnp.float32) for _ in range(NBUF)]
        + [pltpu.VMEM((KB,), jnp.float32) for _ in range(NBUF)]
        + [pltpu.SemaphoreType.DMA for _ in range(3 * NBUF)]
    ),
)


def _dense_body(p_ref, x_ref, wg_ref, w1_ref, b1_ref, g_ref, be_ref, w2_ref,
                b2_ref, o_ref):
    agg = p_ref[0] + p_ref[1] - x_ref[...]
    ne = jnp.maximum(jnp.dot(agg, wg_ref[...],
                             preferred_element_type=jnp.float32), 0.0)
    h = jnp.dot(ne, w1_ref[...], preferred_element_type=jnp.float32)
    h = jnp.maximum(h + b1_ref[...], 0.0)
    mean = jnp.mean(h, axis=0, keepdims=True)
    var = jnp.mean((h - mean) * (h - mean), axis=0, keepdims=True)
    hn = (h - mean) / jnp.sqrt(var + 1e-5) * g_ref[...] + be_ref[...]
    out = jnp.dot(hn, w2_ref[...], preferred_element_type=jnp.float32)
    o_ref[...] = jax.nn.sigmoid(out + b2_ref[...])


_dense_call = pl.pallas_call(
    _dense_body,
    out_shape=jax.ShapeDtypeStruct((N, H2), jnp.float32),
)


def _pack_edges(edge_index, edge_time, edge_weight):
    pad = EPW_P - EPW
    dst = edge_index[1].reshape(NW, EPW)
    src = edge_index[0].reshape(NW, EPW)
    et = edge_time.reshape(NW, EPW)
    ew = edge_weight.reshape(NW, EPW)
    cfg = [(0, 0), (0, pad)]
    dst, src, et, ew = [jnp.pad(a, cfg).reshape(NW, NB, KB)
                        for a in (dst, src, et, ew)]
    pki = jnp.stack([dst, src], axis=2).reshape(NW * NB, 2, KB)
    pkf = jnp.stack([et, ew], axis=2).reshape(NW * NB, 2, KB)
    return pki, pkf


def kernel(x, edge_index, edge_time, node_time, edge_weight,
           W_g, W1, b1, gamma, beta, W2, b2):
    pki, pkf = _pack_edges(edge_index, edge_time, edge_weight)
    partials = _sc_aggregate(x, pki, pkf, node_time)
    return _dense_call(partials, x, W_g, W1, b1.reshape(1, H1),
                       gamma.reshape(1, H1), beta.reshape(1, H1), W2,
                       b2.reshape(1, H2))
